# baseline (speedup 1.0000x reference)
import functools

import jax
import jax.numpy as jnp
from jax import lax
from jax.experimental import pallas as pl
from jax.experimental.pallas import tpu as pltpu
from jax.experimental.pallas import tpu_sc as plsc

N = 10000
H = 128
G = 64
B = 48
CH = 8
NTILES = 32
ROWS_PER_TILE = N // 16
POOL_PER_TILE = 320

_f32 = jnp.float32

_GDN = jax.lax.GatherDimensionNumbers(
    offset_dims=(), collapsed_slice_dims=(0,), start_index_map=(0,))


def _shuffle(v, idx):
    return jax.lax.gather(v, idx[:, None], _GDN, (1,),
                          mode=jax.lax.GatherScatterMode.PROMISE_IN_BOUNDS)


def _lane_allsum(v, iota16):
    for sh in (8, 4, 2, 1):
        v = v + _shuffle(v, jnp.bitwise_xor(iota16, sh))
    return v


@functools.lru_cache(maxsize=None)
def _make_edge_kernel(kind: str, per_tile: int):
    mesh = plsc.VectorSubcoreMesh(core_axis_name="c", subcore_axis_name="s")
    nb = per_tile // B
    is_gat = kind == "gat"
    out_type = [jax.ShapeDtypeStruct((2, N, H), _f32)]
    scratch = [
        pltpu.VMEM_SHARED((N, H), _f32),
        pltpu.VMEM((CH * B,), jnp.int32),
        pltpu.VMEM((CH, B), jnp.int32),
        pltpu.VMEM((CH * B + 16,), _f32),
        pltpu.VMEM((B, H), _f32),
        pltpu.VMEM((B, H), _f32),
        pltpu.VMEM((B, H), _f32),
        pltpu.VMEM((B, H), _f32),
        pltpu.VMEM((H,), _f32),
        pltpu.SemaphoreType.DMA,
        pltpu.SemaphoreType.DMA,
        pltpu.SemaphoreType.DMA,
        pltpu.SemaphoreType.DMA,
    ]
    if is_gat:
        out_type.append(jax.ShapeDtypeStruct((2, N, 16), _f32))
        scratch += [
            pltpu.VMEM_SHARED((N, 16), _f32),
            pltpu.VMEM((B, 16), _f32),
            pltpu.VMEM((B, 16), _f32),
        ]

    @functools.partial(
        pl.kernel,
        out_type=tuple(out_type),
        mesh=mesh,
        scratch_types=scratch,
        compiler_params=pltpu.CompilerParams(use_tc_tiling_on_sc=False),
    )
    def edge_kernel(src_tab, dst_tab, sidx_h, didx_h, msk_h, a_h, *rest):
        if is_gat:
            (out_h, den_h, acc, sidx_c, didx_c, msk_c, hls0, hls1, hrd0, hrd1,
             av, ga0, gb0, ga1, gb1, accd, oden0, oden1) = rest
        else:
            (out_h, acc, sidx_c, didx_c, msk_c, hls0, hls1, hrd0, hrd1,
             av, ga0, gb0, ga1, gb1) = rest
            oden0 = oden1 = accd = None
        c = lax.axis_index("c")
        s = lax.axis_index("s")
        w = c * 16 + s
        zero16 = jnp.zeros((16,), _f32)
        iota16 = lax.iota(jnp.int32, 16)
        zero_idx = jnp.zeros((16,), jnp.int32)

        def zr(i, carry):
            hls0[i // 8, pl.ds((i % 8) * 16, 16)] = zero16
            return carry
        lax.fori_loop(0, B * 8, zr, 0)
        if is_gat:
            def zrd(i, carry):
                oden0[i, pl.ds(0, 16)] = zero16
                return carry
            lax.fori_loop(0, B, zrd, 0)
        row0 = 8 * (78 * s + jnp.minimum(s, 2))
        for rep in range(13):
            pltpu.sync_copy(hls0, acc.at[pl.ds(row0 + rep * B, B)])
            if is_gat:
                pltpu.sync_copy(oden0, accd.at[pl.ds(row0 + rep * B, B)])

        @pl.when(s < 2)
        def _():
            pltpu.sync_copy(hls0.at[pl.ds(0, 8)],
                            acc.at[pl.ds(row0 + 13 * B, 8)])
            if is_gat:
                pltpu.sync_copy(oden0.at[pl.ds(0, 8)],
                                accd.at[pl.ds(row0 + 13 * B, 8)])

        pltpu.sync_copy(a_h, av)
        plsc.subcore_barrier()

        base = w * per_tile
        brow = w * nb

        def load_chunk(t):
            off = base + t * B
            pltpu.sync_copy(sidx_h.at[pl.ds(off, CH * B)], sidx_c)
            pltpu.sync_copy(msk_h.at[pl.ds(off, CH * B)],
                            msk_c.at[pl.ds(0, CH * B)])
            pltpu.sync_copy(didx_h.at[pl.ds(brow + t, CH)], didx_c)

        def issue_gathers(t, hlsX, hrdX, gaX, gbX):
            k = t % CH
            pltpu.async_copy(src_tab.at[sidx_c.at[pl.ds(k * B, B)]], hlsX, gaX)
            pltpu.async_copy(dst_tab.at[didx_c.at[k]], hrdX, gbX)

        load_chunk(0)
        issue_gathers(0, hls0, hrd0, ga0, gb0)

        def do_batch(t, hlsX, hrdX, odenX, gaX, gbX, nxt_bufs,
                     issue_next=True):
            hlsY, hrdY, gaY, gbY = nxt_bufs
            k = t % CH
            kb = k * B
            pltpu.make_async_copy(
                src_tab.at[sidx_c.at[pl.ds(kb, B)]], hlsX, gaX).wait()
            pltpu.make_async_copy(
                dst_tab.at[didx_c.at[k]], hrdX, gbX).wait()

            def edge(j, ecarry):
                acc16 = zero16
                for ch in range(8):
                    u = hlsX[j, pl.ds(ch * 16, 16)]
                    v = hrdX[j, pl.ds(ch * 16, 16)]
                    tt = u + v
                    z = jnp.maximum(tt, 0.2 * tt)
                    acc16 = acc16 + z * av[pl.ds(ch * 16, 16)]
                mv = _shuffle(msk_c[pl.ds(kb + j, 16)], zero_idx)
                ev = _lane_allsum(acc16, iota16)
                ev = jnp.minimum(jnp.maximum(ev, -60.0), 60.0)
                if is_gat:
                    wv = jnp.exp(ev) * mv
                else:
                    t2 = jnp.exp(ev + ev)
                    wv = ((t2 - 1.0) / (t2 + 1.0)) * mv
                for ch in range(8):
                    hlsX[j, pl.ds(ch * 16, 16)] = (
                        hlsX[j, pl.ds(ch * 16, 16)] * wv)
                if is_gat:
                    odenX[j, pl.ds(0, 16)] = wv
                return ecarry

            lax.fori_loop(0, B, edge, 0)
            pltpu.sync_copy(hlsX, acc.at[didx_c.at[k]], add=True)
            if is_gat:
                pltpu.sync_copy(odenX, accd.at[didx_c.at[k]], add=True)

            nxt = t + 1
            if issue_next:

                @pl.when(nxt % CH == 0)
                def _():
                    load_chunk(nxt)

                issue_gathers(nxt, hlsY, hrdY, gaY, gbY)

        def pair(i, carry):
            t0 = i * 2
            do_batch(t0, hls0, hrd0, oden0, ga0, gb0, (hls1, hrd1, ga1, gb1))
            do_batch(t0 + 1, hls1, hrd1, oden1, ga1, gb1,
                     (hls0, hrd0, ga0, gb0))
            return carry

        lax.fori_loop(0, nb // 2 - 1, pair, 0)
        do_batch(nb - 2, hls0, hrd0, oden0, ga0, gb0, (hls1, hrd1, ga1, gb1))
        last = nb - 1
        kl = last % CH
        pltpu.make_async_copy(
            src_tab.at[sidx_c.at[pl.ds(kl * B, B)]], hls1, ga1).wait()
        pltpu.make_async_copy(dst_tab.at[didx_c.at[kl]], hrd1, gb1).wait()

        def edge_last(j, ecarry):
            acc16 = zero16
            for ch in range(8):
                u = hls1[j, pl.ds(ch * 16, 16)]
                v = hrd1[j, pl.ds(ch * 16, 16)]
                tt = u + v
                z = jnp.maximum(tt, 0.2 * tt)
                acc16 = acc16 + z * av[pl.ds(ch * 16, 16)]
            mv = _shuffle(msk_c[pl.ds(kl * B + j, 16)], zero_idx)
            ev = _lane_allsum(acc16, iota16)
            ev = jnp.minimum(jnp.maximum(ev, -60.0), 60.0)
            if is_gat:
                wv = jnp.exp(ev) * mv
            else:
                t2 = jnp.exp(ev + ev)
                wv = ((t2 - 1.0) / (t2 + 1.0)) * mv
            for ch in range(8):
                hls1[j, pl.ds(ch * 16, 16)] = hls1[j, pl.ds(ch * 16, 16)] * wv
            if is_gat:
                oden1[j, pl.ds(0, 16)] = wv
            return ecarry

        lax.fori_loop(0, B, edge_last, 0)
        pltpu.sync_copy(hls1, acc.at[didx_c.at[kl]], add=True)
        if is_gat:
            pltpu.sync_copy(oden1, accd.at[didx_c.at[kl]], add=True)

        plsc.subcore_barrier()

        @pl.when(s < 2)
        def _():
            pltpu.sync_copy(acc.at[pl.ds(row0, 632)],
                            out_h.at[c, pl.ds(row0, 632)])
            if is_gat:
                pltpu.sync_copy(accd.at[pl.ds(row0, 632)],
                                den_h.at[c, pl.ds(row0, 632)])

        @pl.when(s >= 2)
        def _():
            pltpu.sync_copy(acc.at[pl.ds(row0, 624)],
                            out_h.at[c, pl.ds(row0, 624)])
            if is_gat:
                pltpu.sync_copy(accd.at[pl.ds(row0, 624)],
                                den_h.at[c, pl.ds(row0, 624)])

    return edge_kernel


@functools.lru_cache(maxsize=None)
def _make_pool_kernel():
    mesh = plsc.VectorSubcoreMesh(core_axis_name="c", subcore_axis_name="s")

    @functools.partial(
        pl.kernel,
        out_type=(
            jax.ShapeDtypeStruct((NTILES, G, H), _f32),
            jax.ShapeDtypeStruct((NTILES, G, H), _f32),
            jax.ShapeDtypeStruct((NTILES, G, 16), _f32),
        ),
        mesh=mesh,
        scratch_types=[
            pltpu.VMEM((POOL_PER_TILE, H), _f32),
            pltpu.VMEM((POOL_PER_TILE + 16,), jnp.int32),
            pltpu.VMEM((G, H), _f32),
            pltpu.VMEM((G, H), _f32),
            pltpu.VMEM((G, 16), _f32),
        ],
        compiler_params=pltpu.CompilerParams(use_tc_tiling_on_sc=False),
    )
    def pool_kernel(x_h, bat_h, pmax_h, psum_h, pcnt_h,
                    rows, bat, lmax, lsum, lcnt):
        c = lax.axis_index("c")
        s = lax.axis_index("s")
        w = c * 16 + s
        neg = jnp.full((16,), -3.4e38, _f32)
        zero16 = jnp.zeros((16,), _f32)

        def init(i, carry):
            lmax[i // 8, pl.ds((i % 8) * 16, 16)] = neg
            lsum[i // 8, pl.ds((i % 8) * 16, 16)] = zero16
            return carry
        lax.fori_loop(0, G * 8, init, 0)

        def initc(i, carry):
            lcnt[i, pl.ds(0, 16)] = zero16
            return carry
        lax.fori_loop(0, G, initc, 0)

        start = w * POOL_PER_TILE
        nrows = jnp.where(w == NTILES - 1, N - (NTILES - 1) * POOL_PER_TILE,
                          POOL_PER_TILE)

        @pl.when(w < NTILES - 1)
        def _():
            pltpu.sync_copy(x_h.at[pl.ds(start, POOL_PER_TILE)], rows)
            pltpu.sync_copy(bat_h.at[pl.ds(start, POOL_PER_TILE)],
                            bat.at[pl.ds(0, POOL_PER_TILE)])

        last = N - (NTILES - 1) * POOL_PER_TILE

        @pl.when(w == NTILES - 1)
        def _():
            pltpu.sync_copy(x_h.at[pl.ds(start, last)], rows.at[pl.ds(0, last)])
            pltpu.sync_copy(bat_h.at[pl.ds(start, last)], bat.at[pl.ds(0, last)])

        def node(i, carry):
            g = bat[pl.ds(i, 16)][0]
            for ch in range(8):
                r = rows[i, pl.ds(ch * 16, 16)]
                lmax[g, pl.ds(ch * 16, 16)] = jnp.maximum(
                    lmax[g, pl.ds(ch * 16, 16)], r)
                lsum[g, pl.ds(ch * 16, 16)] = lsum[g, pl.ds(ch * 16, 16)] + r
            lcnt[g, pl.ds(0, 16)] = lcnt[g, pl.ds(0, 16)] + 1.0
            return carry

        lax.fori_loop(0, nrows, node, 0)
        pltpu.sync_copy(lmax, pmax_h.at[w])
        pltpu.sync_copy(lsum, psum_h.at[w])
        pltpu.sync_copy(lcnt, pcnt_h.at[w])

    return pool_kernel


_BLK = 400
_GRID = N // _BLK


def _tc_prep(x, wft, bft, wstack):
    K = wstack.shape[0]

    def body(x_ref, wft_ref, bft_ref, ws_ref, h_ref, *t_refs):
        h = jnp.dot(x_ref[...], wft_ref[...],
                    preferred_element_type=_f32) + bft_ref[...]
        h_ref[...] = h
        for k in range(K):
            t_refs[k][...] = jnp.dot(h, ws_ref[k],
                                     preferred_element_type=_f32)

    blk = pl.BlockSpec((_BLK, H), lambda i: (i, 0))
    full_w = pl.BlockSpec((H, H), lambda i: (0, 0))
    return pl.pallas_call(
        body,
        grid=(_GRID,),
        in_specs=[blk, full_w, pl.BlockSpec((1, H), lambda i: (0, 0)),
                  pl.BlockSpec((K, H, H), lambda i: (0, 0, 0))],
        out_specs=[blk] * (1 + K),
        out_shape=[jax.ShapeDtypeStruct((N, H), _f32)] * (1 + K),
    )(x, wft, bft, wstack)


def _tc_combine(xprev, gnum, gden, het_nums, bg, bhs, wstack):
    K = 0 if wstack is None else wstack.shape[0]
    nhet = len(het_nums)

    def body(*refs):
        i = 0
        x_ref = refs[i]; i += 1
        gn_ref = refs[i]; i += 1
        gd_ref = refs[i]; i += 1
        h_refs = refs[i:i + nhet]; i += nhet
        bg_ref = refs[i]; i += 1
        bh_refs = refs[i:i + nhet]; i += nhet
        ws_ref = None
        if K:
            ws_ref = refs[i]; i += 1
        out_ref = refs[i]; i += 1
        t_refs = refs[i:]

        num = gn_ref[0] + gn_ref[1]
        den = gd_ref[0][:, 0:1] + gd_ref[1][:, 0:1]
        z = x_ref[...] + jnp.maximum(num / (den + 1e-16) + bg_ref[...], 0.0)
        for j in range(nhet):
            hsum = h_refs[j][0] + h_refs[j][1]
            z = z + jnp.maximum(hsum + bh_refs[j][...], 0.0)
        nrm = jnp.sqrt(jnp.sum(z * z, axis=1, keepdims=True))
        z = z / jnp.maximum(nrm, 1e-12)
        out_ref[...] = z
        for k in range(K):
            t_refs[k][...] = jnp.dot(z, ws_ref[k], preferred_element_type=_f32)

    blk = pl.BlockSpec((_BLK, H), lambda i: (i, 0))
    accblk = pl.BlockSpec((2, _BLK, H), lambda i: (0, i, 0))
    denblk = pl.BlockSpec((2, _BLK, 16), lambda i: (0, i, 0))
    bias = pl.BlockSpec((1, H), lambda i: (0, 0))
    in_specs = [blk, accblk, denblk] + [accblk] * nhet + [bias] + [bias] * nhet
    args = [xprev, gnum, gden] + list(het_nums) + [bg] + list(bhs)
    if K:
        in_specs.append(pl.BlockSpec((K, H, H), lambda i: (0, 0, 0)))
        args.append(wstack)
    return pl.pallas_call(
        body,
        grid=(_GRID,),
        in_specs=in_specs,
        out_specs=[blk] * (1 + K),
        out_shape=[jax.ShapeDtypeStruct((N, H), _f32)] * (1 + K),
    )(*args)


def _tc_head(pools2, pools3, w1, b1, w2, b2, w3p, b3p):
    def body(mx2, s2, c2, mx3, s3, c3, w1r, b1r, w2r, b2r, w3r, b3r, out):
        def readout(mx, sm, ct):
            m = jnp.max(mx[...], axis=0)
            s = jnp.sum(sm[...], axis=0)
            c = jnp.sum(ct[...], axis=0)[:, 0:1]
            gmp = jnp.where(c > 0.0, m, 0.0)
            gap = s / jnp.maximum(c, 1.0)
            return gmp, gap

        gmp2, gap2 = readout(mx2, s2, c2)
        gmp3, gap3 = readout(mx3, s3, c3)
        xc = jnp.concatenate([gmp2, gap2, gmp3, gap3], axis=1)
        o = jnp.maximum(jnp.dot(xc, w1r[...], preferred_element_type=_f32)
                        + b1r[...], 0.0)
        o = jnp.maximum(jnp.dot(o, w2r[...], preferred_element_type=_f32)
                        + b2r[...], 0.0)
        logits = jnp.dot(o, w3r[...], preferred_element_type=_f32) + b3r[...]
        mx = jnp.max(logits, axis=1, keepdims=True)
        ls = logits - mx
        out[...] = ls - jnp.log(jnp.sum(jnp.exp(ls), axis=1, keepdims=True))

    return pl.pallas_call(
        body,
        out_shape=jax.ShapeDtypeStruct((G, H), _f32),
    )(*pools2, *pools3, w1, b1, w2, b2, w3p, b3p)


def _pad_edges(idx, mask, per_tile):
    total = NTILES * per_tile
    e = idx.shape[1]
    src = jnp.pad(idx[0], (0, total - e))
    dst = jnp.pad(idx[1], (0, total - e)).reshape(total // B, B)
    m = jnp.pad(mask.astype(_f32), (0, total - e))
    return src, dst, m


def _per_tile(e):
    per = -(-e // NTILES)
    return -(-per // (B * CH)) * (B * CH)


def kernel(x, edge_index, two_hop_edge_index, batch, homophily_mask,
           heterophily_mask, hom_hom_mask, het_het_mask, mixed_mask,
           last_epoch, params):
    p = params
    pt1 = _per_tile(edge_index.shape[1])
    pt2 = _per_tile(two_hop_edge_index.shape[1])
    s1, d1, m_hom = _pad_edges(edge_index, homophily_mask, pt1)
    _, _, m_het = _pad_edges(edge_index, heterophily_mask, pt1)
    s2, d2, m_hh = _pad_edges(two_hop_edge_index, hom_hom_mask, pt2)
    _, _, m_tt = _pad_edges(two_hop_edge_index, het_het_mask, pt2)
    _, _, m_mm = _pad_edges(two_hop_edge_index, mixed_mask, pt2)

    gat1_k = _make_edge_kernel("gat", pt1)
    het1_k = _make_edge_kernel("het", pt1)
    gat2_k = _make_edge_kernel("gat", pt2)
    het2_k = _make_edge_kernel("het", pt2)
    pool_k = _make_pool_kernel()

    w4 = jnp.stack([p["graph_hom"]["Wl"], p["graph_hom"]["Wr"],
                    p["graph_het"]["Wl"], p["graph_het"]["Wr"]])
    h, hl_g, hr_g, hl_h, hr_h = _tc_prep(x, p["ft"]["W"], p["ft"]["b"][None],
                                         w4)

    gnum1, gden1 = gat1_k(hl_g, hr_g, s1, d1, m_hom, p["graph_hom"]["a"])
    (hnum1,) = het1_k(hl_h, hr_h, s1, d1, m_het, p["graph_het"]["a"])

    def wstack6(i):
        return jnp.stack([p["hom"][i]["Wl"], p["hom"][i]["Wr"],
                          p["het"][i]["Wl"], p["het"][i]["Wr"],
                          p["mixed"][i]["Wl"], p["mixed"][i]["Wr"]])

    x1, t0, t1, t2, t3, t4, t5 = _tc_combine(
        h, gnum1, gden1, [hnum1], p["graph_hom"]["b"][None],
        [p["graph_het"]["b"][None]], wstack6(0))

    gnum_a, gden_a = gat2_k(t0, t1, s2, d2, m_hh, p["hom"][0]["a"])
    (hnum_a,) = het2_k(t2, t3, s2, d2, m_tt, p["het"][0]["a"])
    (mnum_a,) = het2_k(t4, t5, s2, d2, m_mm, p["mixed"][0]["a"])

    x2, u0, u1, u2, u3, u4, u5 = _tc_combine(
        x1, gnum_a, gden_a, [hnum_a, mnum_a], p["hom"][0]["b"][None],
        [p["het"][0]["b"][None], p["mixed"][0]["b"][None]], wstack6(1))

    gnum_b, gden_b = gat2_k(u0, u1, s2, d2, m_hh, p["hom"][1]["a"])
    (hnum_b,) = het2_k(u2, u3, s2, d2, m_tt, p["het"][1]["a"])
    (mnum_b,) = het2_k(u4, u5, s2, d2, m_mm, p["mixed"][1]["a"])

    (x3,) = _tc_combine(
        x2, gnum_b, gden_b, [hnum_b, mnum_b], p["hom"][1]["b"][None],
        [p["het"][1]["b"][None], p["mixed"][1]["b"][None]], None)

    pools2 = pool_k(x2, batch)
    pools3 = pool_k(x3, batch)

    w3 = p["lin3"]["W"]
    c_out = w3.shape[1]
    w3p = jnp.pad(w3, ((0, 0), (0, H - c_out)))
    b3p = jnp.pad(p["lin3"]["b"], (0, H - c_out),
                  constant_values=-1e30)[None]
    out = _tc_head(pools2, pools3, p["lin1"]["W"], p["lin1"]["b"][None],
                   p["lin2"]["W"], p["lin2"]["b"][None], w3p, b3p)
    return out[:, :c_out]


# speedup vs baseline: 1.5404x; 1.5404x over previous
import functools

import jax
import jax.numpy as jnp
from jax import lax
from jax.experimental import pallas as pl
from jax.experimental.pallas import tpu as pltpu
from jax.experimental.pallas import tpu_sc as plsc

N = 10000
H = 128
G = 64
B = 48
CH = 8
NTILES = 32
ROWS_PER_TILE = N // 16
POOL_PER_TILE = 320

_f32 = jnp.float32

_GDN = jax.lax.GatherDimensionNumbers(
    offset_dims=(), collapsed_slice_dims=(0,), start_index_map=(0,))


def _shuffle(v, idx):
    return jax.lax.gather(v, idx[:, None], _GDN, (1,),
                          mode=jax.lax.GatherScatterMode.PROMISE_IN_BOUNDS)


def _lane_allsum(v, iota16):
    for sh in (8, 4, 2, 1):
        v = v + _shuffle(v, jnp.bitwise_xor(iota16, sh))
    return v


@functools.lru_cache(maxsize=None)
def _make_edge_kernel(kind: str, per_tile: int):
    mesh = plsc.VectorSubcoreMesh(core_axis_name="c", subcore_axis_name="s")
    nb = per_tile // B
    is_gat = kind == "gat"
    out_type = [jax.ShapeDtypeStruct((2, N, H), _f32)]
    scratch = [
        pltpu.VMEM_SHARED((N, H), _f32),
        pltpu.VMEM((2, CH * B), jnp.int32),
        pltpu.VMEM((2, CH, B), jnp.int32),
        pltpu.VMEM((2, CH * B + 16), _f32),
        pltpu.VMEM((B, H), _f32),
        pltpu.VMEM((B, H), _f32),
        pltpu.VMEM((B, H), _f32),
        pltpu.VMEM((B, H), _f32),
        pltpu.VMEM((H,), _f32),
        pltpu.SemaphoreType.DMA,
        pltpu.SemaphoreType.DMA,
        pltpu.SemaphoreType.DMA,
        pltpu.SemaphoreType.DMA,
    ]
    if is_gat:
        out_type.append(jax.ShapeDtypeStruct((2, N, 16), _f32))
        scratch += [
            pltpu.VMEM_SHARED((N, 16), _f32),
            pltpu.VMEM((B, 16), _f32),
            pltpu.VMEM((B, 16), _f32),
        ]

    @functools.partial(
        pl.kernel,
        out_type=tuple(out_type),
        mesh=mesh,
        scratch_types=scratch,
        compiler_params=pltpu.CompilerParams(use_tc_tiling_on_sc=False),
    )
    def edge_kernel(src_tab, dst_tab, sidx_h, didx_h, msk_h, a_h, *rest):
        if is_gat:
            (out_h, den_h, acc, sidx_c, didx_c, msk_c, hls0, hls1, hrd0, hrd1,
             av, ga0, gb0, ga1, gb1, accd, oden0, oden1) = rest
        else:
            (out_h, acc, sidx_c, didx_c, msk_c, hls0, hls1, hrd0, hrd1,
             av, ga0, gb0, ga1, gb1) = rest
            oden0 = oden1 = accd = None
        c = lax.axis_index("c")
        s = lax.axis_index("s")
        w = c * 16 + s
        zero16 = jnp.zeros((16,), _f32)
        iota16 = lax.iota(jnp.int32, 16)
        zero_idx = jnp.zeros((16,), jnp.int32)

        def zr(i, carry):
            hls0[i // 8, pl.ds((i % 8) * 16, 16)] = zero16
            return carry
        lax.fori_loop(0, B * 8, zr, 0)
        if is_gat:
            def zrd(i, carry):
                oden0[i, pl.ds(0, 16)] = zero16
                return carry
            lax.fori_loop(0, B, zrd, 0)
        row0 = 8 * (78 * s + jnp.minimum(s, 2))
        for rep in range(13):
            pltpu.sync_copy(hls0, acc.at[pl.ds(row0 + rep * B, B)])
            if is_gat:
                pltpu.sync_copy(oden0, accd.at[pl.ds(row0 + rep * B, B)])

        @pl.when(s < 2)
        def _():
            pltpu.sync_copy(hls0.at[pl.ds(0, 8)],
                            acc.at[pl.ds(row0 + 13 * B, 8)])
            if is_gat:
                pltpu.sync_copy(oden0.at[pl.ds(0, 8)],
                                accd.at[pl.ds(row0 + 13 * B, 8)])

        pltpu.sync_copy(a_h, av)
        plsc.subcore_barrier()

        base = w * per_tile
        brow = w * nb

        def load_chunk(t):
            pch = (t // CH) % 2
            off = base + t * B
            pltpu.sync_copy(sidx_h.at[pl.ds(off, CH * B)], sidx_c.at[pch])
            pltpu.sync_copy(msk_h.at[pl.ds(off, CH * B)],
                            msk_c.at[pch, pl.ds(0, CH * B)])
            pltpu.sync_copy(didx_h.at[pl.ds(brow + t, CH)], didx_c.at[pch])

        def issue_gathers(t, hlsX, hrdX, gaX, gbX):
            pch = (t // CH) % 2
            k = t % CH
            pltpu.async_copy(
                src_tab.at[sidx_c.at[pch, pl.ds(k * B, B)]], hlsX, gaX)
            pltpu.async_copy(dst_tab.at[didx_c.at[pch, k]], hrdX, gbX)

        def wait_gathers(t, hlsX, hrdX, gaX, gbX):
            pch = (t // CH) % 2
            k = t % CH
            pltpu.make_async_copy(
                src_tab.at[sidx_c.at[pch, pl.ds(k * B, B)]], hlsX, gaX).wait()
            pltpu.make_async_copy(
                dst_tab.at[didx_c.at[pch, k]], hrdX, gbX).wait()

        load_chunk(0)
        issue_gathers(0, hls0, hrd0, ga0, gb0)

        def compute_batch(t, hlsX, hrdX, odenX):
            pch = (t // CH) % 2
            kb = (t % CH) * B

            def edge(j, ecarry):
                acc16 = zero16
                for ch in range(8):
                    u = hlsX[j, pl.ds(ch * 16, 16)]
                    v = hrdX[j, pl.ds(ch * 16, 16)]
                    tt = u + v
                    z = jnp.maximum(tt, 0.2 * tt)
                    acc16 = acc16 + z * av[pl.ds(ch * 16, 16)]
                mv = _shuffle(msk_c[pch, pl.ds(kb + j, 16)], zero_idx)
                ev = _lane_allsum(acc16, iota16)
                ev = jnp.minimum(jnp.maximum(ev, -60.0), 60.0)
                if is_gat:
                    wv = jnp.exp(ev) * mv
                else:
                    t2 = jnp.exp(ev + ev)
                    wv = ((t2 - 1.0) / (t2 + 1.0)) * mv
                for ch in range(8):
                    hlsX[j, pl.ds(ch * 16, 16)] = (
                        hlsX[j, pl.ds(ch * 16, 16)] * wv)
                if is_gat:
                    odenX[j, pl.ds(0, 16)] = wv
                return ecarry

            lax.fori_loop(0, B, edge, 0)

        def scatter_batch(t, hlsX, odenX):
            pch = (t // CH) % 2
            k = t % CH
            pltpu.sync_copy(hlsX, acc.at[didx_c.at[pch, k]], add=True)
            if is_gat:
                pltpu.sync_copy(odenX, accd.at[didx_c.at[pch, k]], add=True)

        def do_batch(t, cur, nxt_bufs, issue_next=True):
            hlsX, hrdX, odenX, gaX, gbX = cur
            hlsY, hrdY, odenY, gaY, gbY = nxt_bufs
            wait_gathers(t, hlsX, hrdX, gaX, gbX)
            if issue_next:
                nxt = t + 1

                @pl.when(nxt % CH == 0)
                def _():
                    load_chunk(nxt)

                issue_gathers(nxt, hlsY, hrdY, gaY, gbY)
            compute_batch(t, hlsX, hrdX, odenX)
            scatter_batch(t, hlsX, odenX)

        buf0 = (hls0, hrd0, oden0, ga0, gb0)
        buf1 = (hls1, hrd1, oden1, ga1, gb1)

        def pair(i, carry):
            t0 = i * 2
            do_batch(t0, buf0, buf1)
            do_batch(t0 + 1, buf1, buf0)
            return carry

        lax.fori_loop(0, nb // 2 - 1, pair, 0)
        do_batch(nb - 2, buf0, buf1)
        do_batch(nb - 1, buf1, buf0, issue_next=False)

        plsc.subcore_barrier()

        @pl.when(s < 2)
        def _():
            pltpu.sync_copy(acc.at[pl.ds(row0, 632)],
                            out_h.at[c, pl.ds(row0, 632)])
            if is_gat:
                pltpu.sync_copy(accd.at[pl.ds(row0, 632)],
                                den_h.at[c, pl.ds(row0, 632)])

        @pl.when(s >= 2)
        def _():
            pltpu.sync_copy(acc.at[pl.ds(row0, 624)],
                            out_h.at[c, pl.ds(row0, 624)])
            if is_gat:
                pltpu.sync_copy(accd.at[pl.ds(row0, 624)],
                                den_h.at[c, pl.ds(row0, 624)])

    return edge_kernel


@functools.lru_cache(maxsize=None)
def _make_pool_kernel():
    mesh = plsc.VectorSubcoreMesh(core_axis_name="c", subcore_axis_name="s")

    @functools.partial(
        pl.kernel,
        out_type=(
            jax.ShapeDtypeStruct((NTILES, G, H), _f32),
            jax.ShapeDtypeStruct((NTILES, G, H), _f32),
            jax.ShapeDtypeStruct((NTILES, G, 16), _f32),
        ),
        mesh=mesh,
        scratch_types=[
            pltpu.VMEM((POOL_PER_TILE, H), _f32),
            pltpu.VMEM((POOL_PER_TILE + 16,), jnp.int32),
            pltpu.VMEM((G, H), _f32),
            pltpu.VMEM((G, H), _f32),
            pltpu.VMEM((G, 16), _f32),
        ],
        compiler_params=pltpu.CompilerParams(use_tc_tiling_on_sc=False),
    )
    def pool_kernel(x_h, bat_h, pmax_h, psum_h, pcnt_h,
                    rows, bat, lmax, lsum, lcnt):
        c = lax.axis_index("c")
        s = lax.axis_index("s")
        w = c * 16 + s
        neg = jnp.full((16,), -3.4e38, _f32)
        zero16 = jnp.zeros((16,), _f32)

        def init(i, carry):
            lmax[i // 8, pl.ds((i % 8) * 16, 16)] = neg
            lsum[i // 8, pl.ds((i % 8) * 16, 16)] = zero16
            return carry
        lax.fori_loop(0, G * 8, init, 0)

        def initc(i, carry):
            lcnt[i, pl.ds(0, 16)] = zero16
            return carry
        lax.fori_loop(0, G, initc, 0)

        start = w * POOL_PER_TILE
        nrows = jnp.where(w == NTILES - 1, N - (NTILES - 1) * POOL_PER_TILE,
                          POOL_PER_TILE)

        @pl.when(w < NTILES - 1)
        def _():
            pltpu.sync_copy(x_h.at[pl.ds(start, POOL_PER_TILE)], rows)
            pltpu.sync_copy(bat_h.at[pl.ds(start, POOL_PER_TILE)],
                            bat.at[pl.ds(0, POOL_PER_TILE)])

        last = N - (NTILES - 1) * POOL_PER_TILE

        @pl.when(w == NTILES - 1)
        def _():
            pltpu.sync_copy(x_h.at[pl.ds(start, last)], rows.at[pl.ds(0, last)])
            pltpu.sync_copy(bat_h.at[pl.ds(start, last)], bat.at[pl.ds(0, last)])

        def node(i, carry):
            g = bat[pl.ds(i, 16)][0]
            for ch in range(8):
                r = rows[i, pl.ds(ch * 16, 16)]
                lmax[g, pl.ds(ch * 16, 16)] = jnp.maximum(
                    lmax[g, pl.ds(ch * 16, 16)], r)
                lsum[g, pl.ds(ch * 16, 16)] = lsum[g, pl.ds(ch * 16, 16)] + r
            lcnt[g, pl.ds(0, 16)] = lcnt[g, pl.ds(0, 16)] + 1.0
            return carry

        lax.fori_loop(0, nrows, node, 0)
        pltpu.sync_copy(lmax, pmax_h.at[w])
        pltpu.sync_copy(lsum, psum_h.at[w])
        pltpu.sync_copy(lcnt, pcnt_h.at[w])

    return pool_kernel


_BLK = 400
_GRID = N // _BLK


def _tc_prep(x, wft, bft, wstack):
    K = wstack.shape[0]

    def body(x_ref, wft_ref, bft_ref, ws_ref, h_ref, *t_refs):
        h = jnp.dot(x_ref[...], wft_ref[...],
                    preferred_element_type=_f32) + bft_ref[...]
        h_ref[...] = h
        for k in range(K):
            t_refs[k][...] = jnp.dot(h, ws_ref[k],
                                     preferred_element_type=_f32)

    blk = pl.BlockSpec((_BLK, H), lambda i: (i, 0))
    full_w = pl.BlockSpec((H, H), lambda i: (0, 0))
    return pl.pallas_call(
        body,
        grid=(_GRID,),
        in_specs=[blk, full_w, pl.BlockSpec((1, H), lambda i: (0, 0)),
                  pl.BlockSpec((K, H, H), lambda i: (0, 0, 0))],
        out_specs=[blk] * (1 + K),
        out_shape=[jax.ShapeDtypeStruct((N, H), _f32)] * (1 + K),
    )(x, wft, bft, wstack)


def _tc_combine(xprev, gnum, gden, het_nums, bg, bhs, wstack):
    K = 0 if wstack is None else wstack.shape[0]
    nhet = len(het_nums)

    def body(*refs):
        i = 0
        x_ref = refs[i]; i += 1
        gn_ref = refs[i]; i += 1
        gd_ref = refs[i]; i += 1
        h_refs = refs[i:i + nhet]; i += nhet
        bg_ref = refs[i]; i += 1
        bh_refs = refs[i:i + nhet]; i += nhet
        ws_ref = None
        if K:
            ws_ref = refs[i]; i += 1
        out_ref = refs[i]; i += 1
        t_refs = refs[i:]

        num = gn_ref[0] + gn_ref[1]
        den = gd_ref[0][:, 0:1] + gd_ref[1][:, 0:1]
        z = x_ref[...] + jnp.maximum(num / (den + 1e-16) + bg_ref[...], 0.0)
        for j in range(nhet):
            hsum = h_refs[j][0] + h_refs[j][1]
            z = z + jnp.maximum(hsum + bh_refs[j][...], 0.0)
        nrm = jnp.sqrt(jnp.sum(z * z, axis=1, keepdims=True))
        z = z / jnp.maximum(nrm, 1e-12)
        out_ref[...] = z
        for k in range(K):
            t_refs[k][...] = jnp.dot(z, ws_ref[k], preferred_element_type=_f32)

    blk = pl.BlockSpec((_BLK, H), lambda i: (i, 0))
    accblk = pl.BlockSpec((2, _BLK, H), lambda i: (0, i, 0))
    denblk = pl.BlockSpec((2, _BLK, 16), lambda i: (0, i, 0))
    bias = pl.BlockSpec((1, H), lambda i: (0, 0))
    in_specs = [blk, accblk, denblk] + [accblk] * nhet + [bias] + [bias] * nhet
    args = [xprev, gnum, gden] + list(het_nums) + [bg] + list(bhs)
    if K:
        in_specs.append(pl.BlockSpec((K, H, H), lambda i: (0, 0, 0)))
        args.append(wstack)
    return pl.pallas_call(
        body,
        grid=(_GRID,),
        in_specs=in_specs,
        out_specs=[blk] * (1 + K),
        out_shape=[jax.ShapeDtypeStruct((N, H), _f32)] * (1 + K),
    )(*args)


def _tc_head(pools2, pools3, w1, b1, w2, b2, w3p, b3p):
    def body(mx2, s2, c2, mx3, s3, c3, w1r, b1r, w2r, b2r, w3r, b3r, out):
        def readout(mx, sm, ct):
            m = jnp.max(mx[...], axis=0)
            s = jnp.sum(sm[...], axis=0)
            c = jnp.sum(ct[...], axis=0)[:, 0:1]
            gmp = jnp.where(c > 0.0, m, 0.0)
            gap = s / jnp.maximum(c, 1.0)
            return gmp, gap

        gmp2, gap2 = readout(mx2, s2, c2)
        gmp3, gap3 = readout(mx3, s3, c3)
        xc = jnp.concatenate([gmp2, gap2, gmp3, gap3], axis=1)
        o = jnp.maximum(jnp.dot(xc, w1r[...], preferred_element_type=_f32)
                        + b1r[...], 0.0)
        o = jnp.maximum(jnp.dot(o, w2r[...], preferred_element_type=_f32)
                        + b2r[...], 0.0)
        logits = jnp.dot(o, w3r[...], preferred_element_type=_f32) + b3r[...]
        mx = jnp.max(logits, axis=1, keepdims=True)
        ls = logits - mx
        out[...] = ls - jnp.log(jnp.sum(jnp.exp(ls), axis=1, keepdims=True))

    return pl.pallas_call(
        body,
        out_shape=jax.ShapeDtypeStruct((G, H), _f32),
    )(*pools2, *pools3, w1, b1, w2, b2, w3p, b3p)


def _pad_edges(idx, mask, per_tile):
    total = NTILES * per_tile
    e = idx.shape[1]
    src = jnp.pad(idx[0], (0, total - e))
    dst = jnp.pad(idx[1], (0, total - e)).reshape(total // B, B)
    m = jnp.pad(mask.astype(_f32), (0, total - e))
    return src, dst, m


def _per_tile(e):
    per = -(-e // NTILES)
    return -(-per // (B * CH)) * (B * CH)


def kernel(x, edge_index, two_hop_edge_index, batch, homophily_mask,
           heterophily_mask, hom_hom_mask, het_het_mask, mixed_mask,
           last_epoch, params):
    p = params
    pt1 = _per_tile(edge_index.shape[1])
    pt2 = _per_tile(two_hop_edge_index.shape[1])
    s1, d1, m_hom = _pad_edges(edge_index, homophily_mask, pt1)
    _, _, m_het = _pad_edges(edge_index, heterophily_mask, pt1)
    s2, d2, m_hh = _pad_edges(two_hop_edge_index, hom_hom_mask, pt2)
    _, _, m_tt = _pad_edges(two_hop_edge_index, het_het_mask, pt2)
    _, _, m_mm = _pad_edges(two_hop_edge_index, mixed_mask, pt2)

    gat1_k = _make_edge_kernel("gat", pt1)
    het1_k = _make_edge_kernel("het", pt1)
    gat2_k = _make_edge_kernel("gat", pt2)
    het2_k = _make_edge_kernel("het", pt2)
    pool_k = _make_pool_kernel()

    w4 = jnp.stack([p["graph_hom"]["Wl"], p["graph_hom"]["Wr"],
                    p["graph_het"]["Wl"], p["graph_het"]["Wr"]])
    h, hl_g, hr_g, hl_h, hr_h = _tc_prep(x, p["ft"]["W"], p["ft"]["b"][None],
                                         w4)

    gnum1, gden1 = gat1_k(hl_g, hr_g, s1, d1, m_hom, p["graph_hom"]["a"])
    (hnum1,) = het1_k(hl_h, hr_h, s1, d1, m_het, p["graph_het"]["a"])

    def wstack6(i):
        return jnp.stack([p["hom"][i]["Wl"], p["hom"][i]["Wr"],
                          p["het"][i]["Wl"], p["het"][i]["Wr"],
                          p["mixed"][i]["Wl"], p["mixed"][i]["Wr"]])

    x1, t0, t1, t2, t3, t4, t5 = _tc_combine(
        h, gnum1, gden1, [hnum1], p["graph_hom"]["b"][None],
        [p["graph_het"]["b"][None]], wstack6(0))

    gnum_a, gden_a = gat2_k(t0, t1, s2, d2, m_hh, p["hom"][0]["a"])
    (hnum_a,) = het2_k(t2, t3, s2, d2, m_tt, p["het"][0]["a"])
    (mnum_a,) = het2_k(t4, t5, s2, d2, m_mm, p["mixed"][0]["a"])

    x2, u0, u1, u2, u3, u4, u5 = _tc_combine(
        x1, gnum_a, gden_a, [hnum_a, mnum_a], p["hom"][0]["b"][None],
        [p["het"][0]["b"][None], p["mixed"][0]["b"][None]], wstack6(1))

    gnum_b, gden_b = gat2_k(u0, u1, s2, d2, m_hh, p["hom"][1]["a"])
    (hnum_b,) = het2_k(u2, u3, s2, d2, m_tt, p["het"][1]["a"])
    (mnum_b,) = het2_k(u4, u5, s2, d2, m_mm, p["mixed"][1]["a"])

    (x3,) = _tc_combine(
        x2, gnum_b, gden_b, [hnum_b, mnum_b], p["hom"][1]["b"][None],
        [p["het"][1]["b"][None], p["mixed"][1]["b"][None]], None)

    pools2 = pool_k(x2, batch)
    pools3 = pool_k(x3, batch)

    w3 = p["lin3"]["W"]
    c_out = w3.shape[1]
    w3p = jnp.pad(w3, ((0, 0), (0, H - c_out)))
    b3p = jnp.pad(p["lin3"]["b"], (0, H - c_out),
                  constant_values=-1e30)[None]
    out = _tc_head(pools2, pools3, p["lin1"]["W"], p["lin1"]["b"][None],
                   p["lin2"]["W"], p["lin2"]["b"][None], w3p, b3p)
    return out[:, :c_out]


# speedup vs baseline: 1.7329x; 1.1249x over previous
import functools

import numpy as np

import jax
import jax.numpy as jnp
from jax import lax
from jax.experimental import pallas as pl
from jax.experimental.pallas import tpu as pltpu
from jax.experimental.pallas import tpu_sc as plsc

N = 10000
H = 128
G = 64
B = 48
CH = 8
NTILES = 32
ROWS_PER_TILE = N // 16
POOL_PER_TILE = 320

_f32 = jnp.float32

_GDN = jax.lax.GatherDimensionNumbers(
    offset_dims=(), collapsed_slice_dims=(0,), start_index_map=(0,))


def _shuffle(v, idx):
    return jax.lax.gather(v, idx[:, None], _GDN, (1,),
                          mode=jax.lax.GatherScatterMode.PROMISE_IN_BOUNDS)


def _deinterleave(a):
    return a.reshape(4, 16, 2).transpose(0, 2, 1).reshape(-1)


def _perm_matrix():
    pos = np.arange(128)
    g, r, i = pos // 32, (pos % 32) // 16, pos % 16
    perm = 32 * g + 2 * i + r
    pm = np.zeros((128, 128), np.float32)
    pm[pos, perm] = 1.0
    return jnp.asarray(pm)


def _lane_allsum(v, iota16):
    for sh in (8, 4, 2, 1):
        v = v + _shuffle(v, jnp.bitwise_xor(iota16, sh))
    return v


@functools.lru_cache(maxsize=None)
def _make_edge_kernel(kind: str, per_tile: int):
    mesh = plsc.VectorSubcoreMesh(core_axis_name="c", subcore_axis_name="s")
    nb = per_tile // B
    is_gat = kind == "gat"
    out_type = [jax.ShapeDtypeStruct((2, N, H), _f32)]
    scratch = [
        pltpu.VMEM_SHARED((N, H), _f32),
        pltpu.VMEM((2, CH * B), jnp.int32),
        pltpu.VMEM((2, CH, B), jnp.int32),
        pltpu.VMEM((2, CH * B + 16), _f32),
        pltpu.VMEM((B, H), jnp.bfloat16),
        pltpu.VMEM((B, H), jnp.bfloat16),
        pltpu.VMEM((B, H), jnp.bfloat16),
        pltpu.VMEM((B, H), jnp.bfloat16),
        pltpu.VMEM((B, H), _f32),
        pltpu.VMEM((B, H), _f32),
        pltpu.VMEM((H,), _f32),
        pltpu.SemaphoreType.DMA,
        pltpu.SemaphoreType.DMA,
        pltpu.SemaphoreType.DMA,
        pltpu.SemaphoreType.DMA,
    ]
    if is_gat:
        out_type.append(jax.ShapeDtypeStruct((2, N, 16), _f32))
        scratch += [
            pltpu.VMEM_SHARED((N, 16), _f32),
            pltpu.VMEM((B, 16), _f32),
            pltpu.VMEM((B, 16), _f32),
        ]

    @functools.partial(
        pl.kernel,
        out_type=tuple(out_type),
        mesh=mesh,
        scratch_types=scratch,
        compiler_params=pltpu.CompilerParams(use_tc_tiling_on_sc=False,
                                             needs_layout_passes=False),
    )
    def edge_kernel(src_tab, dst_tab, sidx_h, didx_h, msk_h, a_h, *rest):
        if is_gat:
            (out_h, den_h, acc, sidx_c, didx_c, msk_c, hls0, hls1, hrd0, hrd1,
             orow0, orow1, av, ga0, gb0, ga1, gb1, accd, oden0, oden1) = rest
        else:
            (out_h, acc, sidx_c, didx_c, msk_c, hls0, hls1, hrd0, hrd1,
             orow0, orow1, av, ga0, gb0, ga1, gb1) = rest
            oden0 = oden1 = accd = None
        c = lax.axis_index("c")
        s = lax.axis_index("s")
        w = c * 16 + s
        zero16 = jnp.zeros((16,), _f32)
        iota16 = lax.iota(jnp.int32, 16)
        zero_idx = jnp.zeros((16,), jnp.int32)

        def zr(i, carry):
            orow0[i // 8, pl.ds((i % 8) * 16, 16)] = zero16
            return carry
        lax.fori_loop(0, B * 8, zr, 0)
        if is_gat:
            def zrd(i, carry):
                oden0[i, pl.ds(0, 16)] = zero16
                return carry
            lax.fori_loop(0, B, zrd, 0)
        row0 = 8 * (78 * s + jnp.minimum(s, 2))
        for rep in range(13):
            pltpu.sync_copy(orow0, acc.at[pl.ds(row0 + rep * B, B)])
            if is_gat:
                pltpu.sync_copy(oden0, accd.at[pl.ds(row0 + rep * B, B)])

        @pl.when(s < 2)
        def _():
            pltpu.sync_copy(orow0.at[pl.ds(0, 8)],
                            acc.at[pl.ds(row0 + 13 * B, 8)])
            if is_gat:
                pltpu.sync_copy(oden0.at[pl.ds(0, 8)],
                                accd.at[pl.ds(row0 + 13 * B, 8)])

        pltpu.sync_copy(a_h, av)
        plsc.subcore_barrier()

        base = w * per_tile
        brow = w * nb

        def load_chunk(t):
            pch = (t // CH) % 2
            off = base + t * B
            pltpu.sync_copy(sidx_h.at[pl.ds(off, CH * B)], sidx_c.at[pch])
            pltpu.sync_copy(msk_h.at[pl.ds(off, CH * B)],
                            msk_c.at[pch, pl.ds(0, CH * B)])
            pltpu.sync_copy(didx_h.at[pl.ds(brow + t, CH)], didx_c.at[pch])

        def issue_gathers(t, hlsX, hrdX, gaX, gbX):
            pch = (t // CH) % 2
            k = t % CH
            pltpu.async_copy(
                src_tab.at[sidx_c.at[pch, pl.ds(k * B, B)]], hlsX, gaX)
            pltpu.async_copy(dst_tab.at[didx_c.at[pch, k]], hrdX, gbX)

        def wait_gathers(t, hlsX, hrdX, gaX, gbX):
            pch = (t // CH) % 2
            k = t % CH
            pltpu.make_async_copy(
                src_tab.at[sidx_c.at[pch, pl.ds(k * B, B)]], hlsX, gaX).wait()
            pltpu.make_async_copy(
                dst_tab.at[didx_c.at[pch, k]], hrdX, gbX).wait()

        load_chunk(0)
        issue_gathers(0, hls0, hrd0, ga0, gb0)

        def compute_batch(t, hlsX, hrdX, orowX, odenX):
            pch = (t // CH) % 2
            kb = (t % CH) * B

            def edge(j, ecarry):
                acc16 = zero16
                ues = []
                for g in range(4):
                    hb = hlsX[j, pl.ds(g * 32, 32)]
                    vb = hrdX[j, pl.ds(g * 32, 32)]
                    ue, uo = plsc.unpack(hb, format=plsc.PackFormat.INTERLEAVED)
                    ve, vo = plsc.unpack(vb, format=plsc.PackFormat.INTERLEAVED)
                    ues.append((ue, uo))
                    te = ue + ve
                    ze = jnp.maximum(te, 0.2 * te)
                    acc16 = acc16 + ze * av[pl.ds(g * 32, 16)]
                    to = uo + vo
                    zo = jnp.maximum(to, 0.2 * to)
                    acc16 = acc16 + zo * av[pl.ds(g * 32 + 16, 16)]
                mv = _shuffle(msk_c[pch, pl.ds(kb + j, 16)], zero_idx)
                ev = _lane_allsum(acc16, iota16)
                ev = jnp.minimum(jnp.maximum(ev, -60.0), 60.0)
                if is_gat:
                    wv = jnp.exp(ev) * mv
                else:
                    t2 = jnp.exp(ev + ev)
                    wv = ((t2 - 1.0) / (t2 + 1.0)) * mv
                for g in range(4):
                    ue, uo = ues[g]
                    orowX[j, pl.ds(g * 32, 16)] = ue * wv
                    orowX[j, pl.ds(g * 32 + 16, 16)] = uo * wv
                if is_gat:
                    odenX[j, pl.ds(0, 16)] = wv
                return ecarry

            lax.fori_loop(0, B, edge, 0)

        def scatter_batch(t, orowX, odenX):
            pch = (t // CH) % 2
            k = t % CH
            pltpu.sync_copy(orowX, acc.at[didx_c.at[pch, k]], add=True)
            if is_gat:
                pltpu.sync_copy(odenX, accd.at[didx_c.at[pch, k]], add=True)

        def do_batch(t, cur, nxt_bufs, issue_next=True):
            hlsX, hrdX, orowX, odenX, gaX, gbX = cur
            hlsY, hrdY, orowY, odenY, gaY, gbY = nxt_bufs
            wait_gathers(t, hlsX, hrdX, gaX, gbX)
            if issue_next:
                nxt = t + 1

                @pl.when(nxt % CH == 0)
                def _():
                    load_chunk(nxt)

                issue_gathers(nxt, hlsY, hrdY, gaY, gbY)
            compute_batch(t, hlsX, hrdX, orowX, odenX)
            scatter_batch(t, orowX, odenX)

        buf0 = (hls0, hrd0, orow0, oden0, ga0, gb0)
        buf1 = (hls1, hrd1, orow1, oden1, ga1, gb1)

        def pair(i, carry):
            t0 = i * 2
            do_batch(t0, buf0, buf1)
            do_batch(t0 + 1, buf1, buf0)
            return carry

        lax.fori_loop(0, nb // 2 - 1, pair, 0)
        do_batch(nb - 2, buf0, buf1)
        do_batch(nb - 1, buf1, buf0, issue_next=False)

        plsc.subcore_barrier()

        @pl.when(s < 2)
        def _():
            pltpu.sync_copy(acc.at[pl.ds(row0, 632)],
                            out_h.at[c, pl.ds(row0, 632)])
            if is_gat:
                pltpu.sync_copy(accd.at[pl.ds(row0, 632)],
                                den_h.at[c, pl.ds(row0, 632)])

        @pl.when(s >= 2)
        def _():
            pltpu.sync_copy(acc.at[pl.ds(row0, 624)],
                            out_h.at[c, pl.ds(row0, 624)])
            if is_gat:
                pltpu.sync_copy(accd.at[pl.ds(row0, 624)],
                                den_h.at[c, pl.ds(row0, 624)])

    return edge_kernel


@functools.lru_cache(maxsize=None)
def _make_pool_kernel():
    mesh = plsc.VectorSubcoreMesh(core_axis_name="c", subcore_axis_name="s")

    @functools.partial(
        pl.kernel,
        out_type=(
            jax.ShapeDtypeStruct((NTILES, G, H), _f32),
            jax.ShapeDtypeStruct((NTILES, G, H), _f32),
            jax.ShapeDtypeStruct((NTILES, G, 16), _f32),
        ),
        mesh=mesh,
        scratch_types=[
            pltpu.VMEM((POOL_PER_TILE, H), _f32),
            pltpu.VMEM((POOL_PER_TILE + 16,), jnp.int32),
            pltpu.VMEM((G, H), _f32),
            pltpu.VMEM((G, H), _f32),
            pltpu.VMEM((G, 16), _f32),
        ],
        compiler_params=pltpu.CompilerParams(use_tc_tiling_on_sc=False,
                                             needs_layout_passes=False),
    )
    def pool_kernel(x_h, bat_h, pmax_h, psum_h, pcnt_h,
                    rows, bat, lmax, lsum, lcnt):
        c = lax.axis_index("c")
        s = lax.axis_index("s")
        w = c * 16 + s
        neg = jnp.full((16,), -3.4e38, _f32)
        zero16 = jnp.zeros((16,), _f32)

        def init(i, carry):
            lmax[i // 8, pl.ds((i % 8) * 16, 16)] = neg
            lsum[i // 8, pl.ds((i % 8) * 16, 16)] = zero16
            return carry
        lax.fori_loop(0, G * 8, init, 0)

        def initc(i, carry):
            lcnt[i, pl.ds(0, 16)] = zero16
            return carry
        lax.fori_loop(0, G, initc, 0)

        start = w * POOL_PER_TILE
        nrows = jnp.where(w == NTILES - 1, N - (NTILES - 1) * POOL_PER_TILE,
                          POOL_PER_TILE)

        @pl.when(w < NTILES - 1)
        def _():
            pltpu.sync_copy(x_h.at[pl.ds(start, POOL_PER_TILE)], rows)
            pltpu.sync_copy(bat_h.at[pl.ds(start, POOL_PER_TILE)],
                            bat.at[pl.ds(0, POOL_PER_TILE)])

        last = N - (NTILES - 1) * POOL_PER_TILE

        @pl.when(w == NTILES - 1)
        def _():
            pltpu.sync_copy(x_h.at[pl.ds(start, last)], rows.at[pl.ds(0, last)])
            pltpu.sync_copy(bat_h.at[pl.ds(start, last)], bat.at[pl.ds(0, last)])

        def node(i, carry):
            g = bat[pl.ds(i, 16)][0]
            for ch in range(8):
                r = rows[i, pl.ds(ch * 16, 16)]
                lmax[g, pl.ds(ch * 16, 16)] = jnp.maximum(
                    lmax[g, pl.ds(ch * 16, 16)], r)
                lsum[g, pl.ds(ch * 16, 16)] = lsum[g, pl.ds(ch * 16, 16)] + r
            lcnt[g, pl.ds(0, 16)] = lcnt[g, pl.ds(0, 16)] + 1.0
            return carry

        lax.fori_loop(0, nrows, node, 0)
        pltpu.sync_copy(lmax, pmax_h.at[w])
        pltpu.sync_copy(lsum, psum_h.at[w])
        pltpu.sync_copy(lcnt, pcnt_h.at[w])

    return pool_kernel


_BLK = 400
_GRID = N // _BLK


def _tc_prep(x, wft, bft, wstack):
    K = wstack.shape[0]

    def body(x_ref, wft_ref, bft_ref, ws_ref, h_ref, *t_refs):
        h = jnp.dot(x_ref[...], wft_ref[...],
                    preferred_element_type=_f32) + bft_ref[...]
        h_ref[...] = h
        for k in range(K):
            t_refs[k][...] = jnp.dot(
                h, ws_ref[k], preferred_element_type=_f32
            ).astype(jnp.bfloat16)

    blk = pl.BlockSpec((_BLK, H), lambda i: (i, 0))
    full_w = pl.BlockSpec((H, H), lambda i: (0, 0))
    return pl.pallas_call(
        body,
        grid=(_GRID,),
        in_specs=[blk, full_w, pl.BlockSpec((1, H), lambda i: (0, 0)),
                  pl.BlockSpec((K, H, H), lambda i: (0, 0, 0))],
        out_specs=[blk] * (1 + K),
        out_shape=[jax.ShapeDtypeStruct((N, H), _f32)]
        + [jax.ShapeDtypeStruct((N, H), jnp.bfloat16)] * K,
    )(x, wft, bft, wstack)


def _tc_combine(xprev, gnum, gden, het_nums, bg, bhs, wstack, pm):
    K = 0 if wstack is None else wstack.shape[0]
    nhet = len(het_nums)

    def body(*refs):
        i = 0
        x_ref = refs[i]; i += 1
        gn_ref = refs[i]; i += 1
        gd_ref = refs[i]; i += 1
        h_refs = refs[i:i + nhet]; i += nhet
        bg_ref = refs[i]; i += 1
        bh_refs = refs[i:i + nhet]; i += nhet
        pm_ref = refs[i]; i += 1
        ws_ref = None
        if K:
            ws_ref = refs[i]; i += 1
        out_ref = refs[i]; i += 1
        t_refs = refs[i:]

        pmv = pm_ref[...]
        num = jnp.dot(gn_ref[0] + gn_ref[1], pmv, preferred_element_type=_f32)
        den = gd_ref[0][:, 0:1] + gd_ref[1][:, 0:1]
        z = x_ref[...] + jnp.maximum(num / (den + 1e-16) + bg_ref[...], 0.0)
        for j in range(nhet):
            hsum = jnp.dot(h_refs[j][0] + h_refs[j][1], pmv,
                           preferred_element_type=_f32)
            z = z + jnp.maximum(hsum + bh_refs[j][...], 0.0)
        nrm = jnp.sqrt(jnp.sum(z * z, axis=1, keepdims=True))
        z = z / jnp.maximum(nrm, 1e-12)
        out_ref[...] = z
        for k in range(K):
            t_refs[k][...] = jnp.dot(
                z, ws_ref[k], preferred_element_type=_f32
            ).astype(jnp.bfloat16)

    blk = pl.BlockSpec((_BLK, H), lambda i: (i, 0))
    accblk = pl.BlockSpec((2, _BLK, H), lambda i: (0, i, 0))
    denblk = pl.BlockSpec((2, _BLK, 16), lambda i: (0, i, 0))
    bias = pl.BlockSpec((1, H), lambda i: (0, 0))
    full_w = pl.BlockSpec((H, H), lambda i: (0, 0))
    in_specs = ([blk, accblk, denblk] + [accblk] * nhet + [bias]
                + [bias] * nhet + [full_w])
    args = [xprev, gnum, gden] + list(het_nums) + [bg] + list(bhs) + [pm]
    if K:
        in_specs.append(pl.BlockSpec((K, H, H), lambda i: (0, 0, 0)))
        args.append(wstack)
    return pl.pallas_call(
        body,
        grid=(_GRID,),
        in_specs=in_specs,
        out_specs=[blk] * (1 + K),
        out_shape=[jax.ShapeDtypeStruct((N, H), _f32)]
        + [jax.ShapeDtypeStruct((N, H), jnp.bfloat16)] * K,
    )(*args)


def _tc_head(pools2, pools3, w1, b1, w2, b2, w3p, b3p):
    def body(mx2, s2, c2, mx3, s3, c3, w1r, b1r, w2r, b2r, w3r, b3r, out):
        def readout(mx, sm, ct):
            m = jnp.max(mx[...], axis=0)
            s = jnp.sum(sm[...], axis=0)
            c = jnp.sum(ct[...], axis=0)[:, 0:1]
            gmp = jnp.where(c > 0.0, m, 0.0)
            gap = s / jnp.maximum(c, 1.0)
            return gmp, gap

        gmp2, gap2 = readout(mx2, s2, c2)
        gmp3, gap3 = readout(mx3, s3, c3)
        xc = jnp.concatenate([gmp2, gap2, gmp3, gap3], axis=1)
        o = jnp.maximum(jnp.dot(xc, w1r[...], preferred_element_type=_f32)
                        + b1r[...], 0.0)
        o = jnp.maximum(jnp.dot(o, w2r[...], preferred_element_type=_f32)
                        + b2r[...], 0.0)
        logits = jnp.dot(o, w3r[...], preferred_element_type=_f32) + b3r[...]
        mx = jnp.max(logits, axis=1, keepdims=True)
        ls = logits - mx
        out[...] = ls - jnp.log(jnp.sum(jnp.exp(ls), axis=1, keepdims=True))

    return pl.pallas_call(
        body,
        out_shape=jax.ShapeDtypeStruct((G, H), _f32),
    )(*pools2, *pools3, w1, b1, w2, b2, w3p, b3p)


def _pad_edges(idx, mask, per_tile):
    total = NTILES * per_tile
    e = idx.shape[1]
    src = jnp.pad(idx[0], (0, total - e))
    dst = jnp.pad(idx[1], (0, total - e)).reshape(total // B, B)
    m = jnp.pad(mask.astype(_f32), (0, total - e))
    return src, dst, m


def _per_tile(e):
    per = -(-e // NTILES)
    return -(-per // (B * CH)) * (B * CH)


def kernel(x, edge_index, two_hop_edge_index, batch, homophily_mask,
           heterophily_mask, hom_hom_mask, het_het_mask, mixed_mask,
           last_epoch, params):
    p = params
    pt1 = _per_tile(edge_index.shape[1])
    pt2 = _per_tile(two_hop_edge_index.shape[1])
    s1, d1, m_hom = _pad_edges(edge_index, homophily_mask, pt1)
    _, _, m_het = _pad_edges(edge_index, heterophily_mask, pt1)
    s2, d2, m_hh = _pad_edges(two_hop_edge_index, hom_hom_mask, pt2)
    _, _, m_tt = _pad_edges(two_hop_edge_index, het_het_mask, pt2)
    _, _, m_mm = _pad_edges(two_hop_edge_index, mixed_mask, pt2)

    gat1_k = _make_edge_kernel("gat", pt1)
    het1_k = _make_edge_kernel("het", pt1)
    gat2_k = _make_edge_kernel("gat", pt2)
    het2_k = _make_edge_kernel("het", pt2)
    pool_k = _make_pool_kernel()

    w4 = jnp.stack([p["graph_hom"]["Wl"], p["graph_hom"]["Wr"],
                    p["graph_het"]["Wl"], p["graph_het"]["Wr"]])
    h, hl_g, hr_g, hl_h, hr_h = _tc_prep(x, p["ft"]["W"], p["ft"]["b"][None],
                                         w4)

    pm = _perm_matrix()
    gnum1, gden1 = gat1_k(hl_g, hr_g, s1, d1, m_hom,
                          _deinterleave(p["graph_hom"]["a"]))
    (hnum1,) = het1_k(hl_h, hr_h, s1, d1, m_het,
                      _deinterleave(p["graph_het"]["a"]))

    def wstack6(i):
        return jnp.stack([p["hom"][i]["Wl"], p["hom"][i]["Wr"],
                          p["het"][i]["Wl"], p["het"][i]["Wr"],
                          p["mixed"][i]["Wl"], p["mixed"][i]["Wr"]])

    x1, t0, t1, t2, t3, t4, t5 = _tc_combine(
        h, gnum1, gden1, [hnum1], p["graph_hom"]["b"][None],
        [p["graph_het"]["b"][None]], wstack6(0), pm)

    gnum_a, gden_a = gat2_k(t0, t1, s2, d2, m_hh, _deinterleave(p["hom"][0]["a"]))
    (hnum_a,) = het2_k(t2, t3, s2, d2, m_tt, _deinterleave(p["het"][0]["a"]))
    (mnum_a,) = het2_k(t4, t5, s2, d2, m_mm, _deinterleave(p["mixed"][0]["a"]))

    x2, u0, u1, u2, u3, u4, u5 = _tc_combine(
        x1, gnum_a, gden_a, [hnum_a, mnum_a], p["hom"][0]["b"][None],
        [p["het"][0]["b"][None], p["mixed"][0]["b"][None]], wstack6(1), pm)

    gnum_b, gden_b = gat2_k(u0, u1, s2, d2, m_hh, _deinterleave(p["hom"][1]["a"]))
    (hnum_b,) = het2_k(u2, u3, s2, d2, m_tt, _deinterleave(p["het"][1]["a"]))
    (mnum_b,) = het2_k(u4, u5, s2, d2, m_mm, _deinterleave(p["mixed"][1]["a"]))

    (x3,) = _tc_combine(
        x2, gnum_b, gden_b, [hnum_b, mnum_b], p["hom"][1]["b"][None],
        [p["het"][1]["b"][None], p["mixed"][1]["b"][None]], None, pm)

    pools2 = pool_k(x2, batch)
    pools3 = pool_k(x3, batch)

    w3 = p["lin3"]["W"]
    c_out = w3.shape[1]
    w3p = jnp.pad(w3, ((0, 0), (0, H - c_out)))
    b3p = jnp.pad(p["lin3"]["b"], (0, H - c_out),
                  constant_values=-1e30)[None]
    out = _tc_head(pools2, pools3, p["lin1"]["W"], p["lin1"]["b"][None],
                   p["lin2"]["W"], p["lin2"]["b"][None], w3p, b3p)
    return out[:, :c_out]


# speedup vs baseline: 1.7615x; 1.0165x over previous
import functools

import numpy as np

import jax
import jax.numpy as jnp
from jax import lax
from jax.experimental import pallas as pl
from jax.experimental.pallas import tpu as pltpu
from jax.experimental.pallas import tpu_sc as plsc

N = 10000
H = 128
G = 64
B = 64
CH = 8
NTILES = 32
ROWS_PER_TILE = N // 16
POOL_PER_TILE = 320

_f32 = jnp.float32

_GDN = jax.lax.GatherDimensionNumbers(
    offset_dims=(), collapsed_slice_dims=(0,), start_index_map=(0,))


def _shuffle(v, idx):
    return jax.lax.gather(v, idx[:, None], _GDN, (1,),
                          mode=jax.lax.GatherScatterMode.PROMISE_IN_BOUNDS)


def _deinterleave(a):
    return a.reshape(4, 16, 2).transpose(0, 2, 1).reshape(-1)


def _perm_matrix():
    pos = np.arange(128)
    g, r, i = pos // 32, (pos % 32) // 16, pos % 16
    perm = 32 * g + 2 * i + r
    pm = np.zeros((128, 128), np.float32)
    pm[pos, perm] = 1.0
    return jnp.asarray(pm)


def _lane_allsum(v, iota16):
    for sh in (8, 4, 2, 1):
        v = v + _shuffle(v, jnp.bitwise_xor(iota16, sh))
    return v


@functools.lru_cache(maxsize=None)
def _make_edge_kernel(kind: str, per_tile: int):
    mesh = plsc.VectorSubcoreMesh(core_axis_name="c", subcore_axis_name="s")
    nb = per_tile // B
    is_gat = kind == "gat"
    out_type = [jax.ShapeDtypeStruct((2, N, H), _f32)]
    scratch = [
        pltpu.VMEM_SHARED((N, H), _f32),
        pltpu.VMEM((2, CH * B), jnp.int32),
        pltpu.VMEM((2, CH, B), jnp.int32),
        pltpu.VMEM((2, CH * B + 16), _f32),
        pltpu.VMEM((B, H), jnp.bfloat16),
        pltpu.VMEM((B, H), jnp.bfloat16),
        pltpu.VMEM((B, H), jnp.bfloat16),
        pltpu.VMEM((B, H), jnp.bfloat16),
        pltpu.VMEM((B, H), _f32),
        pltpu.VMEM((B, H), _f32),
        pltpu.VMEM((H,), _f32),
        pltpu.SemaphoreType.DMA,
        pltpu.SemaphoreType.DMA,
        pltpu.SemaphoreType.DMA,
        pltpu.SemaphoreType.DMA,
    ]
    if is_gat:
        out_type.append(jax.ShapeDtypeStruct((2, N, 16), _f32))
        scratch += [
            pltpu.VMEM_SHARED((N, 16), _f32),
            pltpu.VMEM((B, 16), _f32),
            pltpu.VMEM((B, 16), _f32),
        ]

    @functools.partial(
        pl.kernel,
        out_type=tuple(out_type),
        mesh=mesh,
        scratch_types=scratch,
        compiler_params=pltpu.CompilerParams(use_tc_tiling_on_sc=False,
                                             needs_layout_passes=False),
    )
    def edge_kernel(src_tab, dst_tab, sidx_h, didx_h, msk_h, a_h, *rest):
        if is_gat:
            (out_h, den_h, acc, sidx_c, didx_c, msk_c, hls0, hls1, hrd0, hrd1,
             orow0, orow1, av, ga0, gb0, ga1, gb1, accd, oden0, oden1) = rest
        else:
            (out_h, acc, sidx_c, didx_c, msk_c, hls0, hls1, hrd0, hrd1,
             orow0, orow1, av, ga0, gb0, ga1, gb1) = rest
            oden0 = oden1 = accd = None
        c = lax.axis_index("c")
        s = lax.axis_index("s")
        w = c * 16 + s
        zero16 = jnp.zeros((16,), _f32)
        iota16 = lax.iota(jnp.int32, 16)
        zero_idx = jnp.zeros((16,), jnp.int32)

        def zr(i, carry):
            orow0[i // 8, pl.ds((i % 8) * 16, 16)] = zero16
            return carry
        lax.fori_loop(0, B * 8, zr, 0)
        if is_gat:
            def zrd(i, carry):
                oden0[i, pl.ds(0, 16)] = zero16
                return carry
            lax.fori_loop(0, B, zrd, 0)
        row0 = 8 * (78 * s + jnp.minimum(s, 2))
        for rep in range(9):
            pltpu.sync_copy(orow0, acc.at[pl.ds(row0 + rep * B, B)])
            if is_gat:
                pltpu.sync_copy(oden0, accd.at[pl.ds(row0 + rep * B, B)])

        @pl.when(s < 2)
        def _():
            pltpu.sync_copy(orow0.at[pl.ds(0, 56)],
                            acc.at[pl.ds(row0 + 9 * B, 56)])
            if is_gat:
                pltpu.sync_copy(oden0.at[pl.ds(0, 56)],
                                accd.at[pl.ds(row0 + 9 * B, 56)])

        @pl.when(s >= 2)
        def _():
            pltpu.sync_copy(orow0.at[pl.ds(0, 48)],
                            acc.at[pl.ds(row0 + 9 * B, 48)])
            if is_gat:
                pltpu.sync_copy(oden0.at[pl.ds(0, 48)],
                                accd.at[pl.ds(row0 + 9 * B, 48)])

        pltpu.sync_copy(a_h, av)
        plsc.subcore_barrier()

        base = w * per_tile
        brow = w * nb

        def load_chunk(t):
            pch = (t // CH) % 2
            off = base + t * B
            pltpu.sync_copy(sidx_h.at[pl.ds(off, CH * B)], sidx_c.at[pch])
            pltpu.sync_copy(msk_h.at[pl.ds(off, CH * B)],
                            msk_c.at[pch, pl.ds(0, CH * B)])
            pltpu.sync_copy(didx_h.at[pl.ds(brow + t, CH)], didx_c.at[pch])

        def issue_gathers(t, hlsX, hrdX, gaX, gbX):
            pch = (t // CH) % 2
            k = t % CH
            pltpu.async_copy(
                src_tab.at[sidx_c.at[pch, pl.ds(k * B, B)]], hlsX, gaX)
            pltpu.async_copy(dst_tab.at[didx_c.at[pch, k]], hrdX, gbX)

        def wait_gathers(t, hlsX, hrdX, gaX, gbX):
            pch = (t // CH) % 2
            k = t % CH
            pltpu.make_async_copy(
                src_tab.at[sidx_c.at[pch, pl.ds(k * B, B)]], hlsX, gaX).wait()
            pltpu.make_async_copy(
                dst_tab.at[didx_c.at[pch, k]], hrdX, gbX).wait()

        load_chunk(0)
        issue_gathers(0, hls0, hrd0, ga0, gb0)

        def compute_batch(t, hlsX, hrdX, orowX, odenX):
            pch = (t // CH) % 2
            kb = (t % CH) * B

            def edge(j, ecarry):
                acc16 = zero16
                ues = []
                for g in range(4):
                    hb = hlsX[j, pl.ds(g * 32, 32)]
                    vb = hrdX[j, pl.ds(g * 32, 32)]
                    ue, uo = plsc.unpack(hb, format=plsc.PackFormat.INTERLEAVED)
                    ve, vo = plsc.unpack(vb, format=plsc.PackFormat.INTERLEAVED)
                    ues.append((ue, uo))
                    te = ue + ve
                    ze = jnp.maximum(te, 0.2 * te)
                    acc16 = acc16 + ze * av[pl.ds(g * 32, 16)]
                    to = uo + vo
                    zo = jnp.maximum(to, 0.2 * to)
                    acc16 = acc16 + zo * av[pl.ds(g * 32 + 16, 16)]
                mv = _shuffle(msk_c[pch, pl.ds(kb + j, 16)], zero_idx)
                ev = _lane_allsum(acc16, iota16)
                ev = jnp.minimum(jnp.maximum(ev, -60.0), 60.0)
                if is_gat:
                    wv = jnp.exp(ev) * mv
                else:
                    t2 = jnp.exp(ev + ev)
                    wv = ((t2 - 1.0) / (t2 + 1.0)) * mv
                for g in range(4):
                    ue, uo = ues[g]
                    orowX[j, pl.ds(g * 32, 16)] = ue * wv
                    orowX[j, pl.ds(g * 32 + 16, 16)] = uo * wv
                if is_gat:
                    odenX[j, pl.ds(0, 16)] = wv
                return ecarry

            lax.fori_loop(0, B, edge, 0)

        def scatter_batch(t, orowX, odenX):
            pch = (t // CH) % 2
            k = t % CH
            pltpu.sync_copy(orowX, acc.at[didx_c.at[pch, k]], add=True)
            if is_gat:
                pltpu.sync_copy(odenX, accd.at[didx_c.at[pch, k]], add=True)

        def do_batch(t, cur, nxt_bufs, issue_next=True):
            hlsX, hrdX, orowX, odenX, gaX, gbX = cur
            hlsY, hrdY, orowY, odenY, gaY, gbY = nxt_bufs
            wait_gathers(t, hlsX, hrdX, gaX, gbX)
            if issue_next:
                nxt = t + 1

                @pl.when(nxt % CH == 0)
                def _():
                    load_chunk(nxt)

                issue_gathers(nxt, hlsY, hrdY, gaY, gbY)
            compute_batch(t, hlsX, hrdX, orowX, odenX)
            scatter_batch(t, orowX, odenX)

        buf0 = (hls0, hrd0, orow0, oden0, ga0, gb0)
        buf1 = (hls1, hrd1, orow1, oden1, ga1, gb1)

        def pair(i, carry):
            t0 = i * 2
            do_batch(t0, buf0, buf1)
            do_batch(t0 + 1, buf1, buf0)
            return carry

        lax.fori_loop(0, nb // 2 - 1, pair, 0)
        do_batch(nb - 2, buf0, buf1)
        do_batch(nb - 1, buf1, buf0, issue_next=False)

        plsc.subcore_barrier()

        @pl.when(s < 2)
        def _():
            pltpu.sync_copy(acc.at[pl.ds(row0, 632)],
                            out_h.at[c, pl.ds(row0, 632)])
            if is_gat:
                pltpu.sync_copy(accd.at[pl.ds(row0, 632)],
                                den_h.at[c, pl.ds(row0, 632)])

        @pl.when(s >= 2)
        def _():
            pltpu.sync_copy(acc.at[pl.ds(row0, 624)],
                            out_h.at[c, pl.ds(row0, 624)])
            if is_gat:
                pltpu.sync_copy(accd.at[pl.ds(row0, 624)],
                                den_h.at[c, pl.ds(row0, 624)])

    return edge_kernel


@functools.lru_cache(maxsize=None)
def _make_pool_kernel():
    mesh = plsc.VectorSubcoreMesh(core_axis_name="c", subcore_axis_name="s")

    @functools.partial(
        pl.kernel,
        out_type=(
            jax.ShapeDtypeStruct((NTILES, G, H), _f32),
            jax.ShapeDtypeStruct((NTILES, G, H), _f32),
            jax.ShapeDtypeStruct((NTILES, G, 16), _f32),
        ),
        mesh=mesh,
        scratch_types=[
            pltpu.VMEM((POOL_PER_TILE, H), _f32),
            pltpu.VMEM((POOL_PER_TILE + 16,), jnp.int32),
            pltpu.VMEM((G, H), _f32),
            pltpu.VMEM((G, H), _f32),
            pltpu.VMEM((G, 16), _f32),
        ],
        compiler_params=pltpu.CompilerParams(use_tc_tiling_on_sc=False,
                                             needs_layout_passes=False),
    )
    def pool_kernel(x_h, bat_h, pmax_h, psum_h, pcnt_h,
                    rows, bat, lmax, lsum, lcnt):
        c = lax.axis_index("c")
        s = lax.axis_index("s")
        w = c * 16 + s
        neg = jnp.full((16,), -3.4e38, _f32)
        zero16 = jnp.zeros((16,), _f32)

        def init(i, carry):
            lmax[i // 8, pl.ds((i % 8) * 16, 16)] = neg
            lsum[i // 8, pl.ds((i % 8) * 16, 16)] = zero16
            return carry
        lax.fori_loop(0, G * 8, init, 0)

        def initc(i, carry):
            lcnt[i, pl.ds(0, 16)] = zero16
            return carry
        lax.fori_loop(0, G, initc, 0)

        start = w * POOL_PER_TILE
        nrows = jnp.where(w == NTILES - 1, N - (NTILES - 1) * POOL_PER_TILE,
                          POOL_PER_TILE)

        @pl.when(w < NTILES - 1)
        def _():
            pltpu.sync_copy(x_h.at[pl.ds(start, POOL_PER_TILE)], rows)
            pltpu.sync_copy(bat_h.at[pl.ds(start, POOL_PER_TILE)],
                            bat.at[pl.ds(0, POOL_PER_TILE)])

        last = N - (NTILES - 1) * POOL_PER_TILE

        @pl.when(w == NTILES - 1)
        def _():
            pltpu.sync_copy(x_h.at[pl.ds(start, last)], rows.at[pl.ds(0, last)])
            pltpu.sync_copy(bat_h.at[pl.ds(start, last)], bat.at[pl.ds(0, last)])

        def node(i, carry):
            g = bat[pl.ds(i, 16)][0]
            for ch in range(8):
                r = rows[i, pl.ds(ch * 16, 16)]
                lmax[g, pl.ds(ch * 16, 16)] = jnp.maximum(
                    lmax[g, pl.ds(ch * 16, 16)], r)
                lsum[g, pl.ds(ch * 16, 16)] = lsum[g, pl.ds(ch * 16, 16)] + r
            lcnt[g, pl.ds(0, 16)] = lcnt[g, pl.ds(0, 16)] + 1.0
            return carry

        lax.fori_loop(0, nrows, node, 0)
        pltpu.sync_copy(lmax, pmax_h.at[w])
        pltpu.sync_copy(lsum, psum_h.at[w])
        pltpu.sync_copy(lcnt, pcnt_h.at[w])

    return pool_kernel


_BLK = 400
_GRID = N // _BLK


def _tc_prep(x, wft, bft, wstack):
    K = wstack.shape[0]

    def body(x_ref, wft_ref, bft_ref, ws_ref, h_ref, *t_refs):
        h = jnp.dot(x_ref[...], wft_ref[...],
                    preferred_element_type=_f32) + bft_ref[...]
        h_ref[...] = h
        for k in range(K):
            t_refs[k][...] = jnp.dot(
                h, ws_ref[k], preferred_element_type=_f32
            ).astype(jnp.bfloat16)

    blk = pl.BlockSpec((_BLK, H), lambda i: (i, 0))
    full_w = pl.BlockSpec((H, H), lambda i: (0, 0))
    return pl.pallas_call(
        body,
        grid=(_GRID,),
        in_specs=[blk, full_w, pl.BlockSpec((1, H), lambda i: (0, 0)),
                  pl.BlockSpec((K, H, H), lambda i: (0, 0, 0))],
        out_specs=[blk] * (1 + K),
        out_shape=[jax.ShapeDtypeStruct((N, H), _f32)]
        + [jax.ShapeDtypeStruct((N, H), jnp.bfloat16)] * K,
    )(x, wft, bft, wstack)


def _tc_combine(xprev, gnum, gden, het_nums, bg, bhs, wstack, pm):
    K = 0 if wstack is None else wstack.shape[0]
    nhet = len(het_nums)

    def body(*refs):
        i = 0
        x_ref = refs[i]; i += 1
        gn_ref = refs[i]; i += 1
        gd_ref = refs[i]; i += 1
        h_refs = refs[i:i + nhet]; i += nhet
        bg_ref = refs[i]; i += 1
        bh_refs = refs[i:i + nhet]; i += nhet
        pm_ref = refs[i]; i += 1
        ws_ref = None
        if K:
            ws_ref = refs[i]; i += 1
        out_ref = refs[i]; i += 1
        t_refs = refs[i:]

        pmv = pm_ref[...]
        num = jnp.dot(gn_ref[0] + gn_ref[1], pmv, preferred_element_type=_f32)
        den = gd_ref[0][:, 0:1] + gd_ref[1][:, 0:1]
        z = x_ref[...] + jnp.maximum(num / (den + 1e-16) + bg_ref[...], 0.0)
        for j in range(nhet):
            hsum = jnp.dot(h_refs[j][0] + h_refs[j][1], pmv,
                           preferred_element_type=_f32)
            z = z + jnp.maximum(hsum + bh_refs[j][...], 0.0)
        nrm = jnp.sqrt(jnp.sum(z * z, axis=1, keepdims=True))
        z = z / jnp.maximum(nrm, 1e-12)
        out_ref[...] = z
        for k in range(K):
            t_refs[k][...] = jnp.dot(
                z, ws_ref[k], preferred_element_type=_f32
            ).astype(jnp.bfloat16)

    blk = pl.BlockSpec((_BLK, H), lambda i: (i, 0))
    accblk = pl.BlockSpec((2, _BLK, H), lambda i: (0, i, 0))
    denblk = pl.BlockSpec((2, _BLK, 16), lambda i: (0, i, 0))
    bias = pl.BlockSpec((1, H), lambda i: (0, 0))
    full_w = pl.BlockSpec((H, H), lambda i: (0, 0))
    in_specs = ([blk, accblk, denblk] + [accblk] * nhet + [bias]
                + [bias] * nhet + [full_w])
    args = [xprev, gnum, gden] + list(het_nums) + [bg] + list(bhs) + [pm]
    if K:
        in_specs.append(pl.BlockSpec((K, H, H), lambda i: (0, 0, 0)))
        args.append(wstack)
    return pl.pallas_call(
        body,
        grid=(_GRID,),
        in_specs=in_specs,
        out_specs=[blk] * (1 + K),
        out_shape=[jax.ShapeDtypeStruct((N, H), _f32)]
        + [jax.ShapeDtypeStruct((N, H), jnp.bfloat16)] * K,
    )(*args)


def _tc_head(pools2, pools3, w1, b1, w2, b2, w3p, b3p):
    def body(mx2, s2, c2, mx3, s3, c3, w1r, b1r, w2r, b2r, w3r, b3r, out):
        def readout(mx, sm, ct):
            m = jnp.max(mx[...], axis=0)
            s = jnp.sum(sm[...], axis=0)
            c = jnp.sum(ct[...], axis=0)[:, 0:1]
            gmp = jnp.where(c > 0.0, m, 0.0)
            gap = s / jnp.maximum(c, 1.0)
            return gmp, gap

        gmp2, gap2 = readout(mx2, s2, c2)
        gmp3, gap3 = readout(mx3, s3, c3)
        xc = jnp.concatenate([gmp2, gap2, gmp3, gap3], axis=1)
        o = jnp.maximum(jnp.dot(xc, w1r[...], preferred_element_type=_f32)
                        + b1r[...], 0.0)
        o = jnp.maximum(jnp.dot(o, w2r[...], preferred_element_type=_f32)
                        + b2r[...], 0.0)
        logits = jnp.dot(o, w3r[...], preferred_element_type=_f32) + b3r[...]
        mx = jnp.max(logits, axis=1, keepdims=True)
        ls = logits - mx
        out[...] = ls - jnp.log(jnp.sum(jnp.exp(ls), axis=1, keepdims=True))

    return pl.pallas_call(
        body,
        out_shape=jax.ShapeDtypeStruct((G, H), _f32),
    )(*pools2, *pools3, w1, b1, w2, b2, w3p, b3p)


def _pad_edges(idx, mask, per_tile):
    total = NTILES * per_tile
    e = idx.shape[1]
    src = jnp.pad(idx[0], (0, total - e))
    dst = jnp.pad(idx[1], (0, total - e)).reshape(total // B, B)
    m = jnp.pad(mask.astype(_f32), (0, total - e))
    return src, dst, m


def _per_tile(e):
    per = -(-e // NTILES)
    return -(-per // (B * CH)) * (B * CH)


def kernel(x, edge_index, two_hop_edge_index, batch, homophily_mask,
           heterophily_mask, hom_hom_mask, het_het_mask, mixed_mask,
           last_epoch, params):
    p = params
    pt1 = _per_tile(edge_index.shape[1])
    pt2 = _per_tile(two_hop_edge_index.shape[1])
    s1, d1, m_hom = _pad_edges(edge_index, homophily_mask, pt1)
    _, _, m_het = _pad_edges(edge_index, heterophily_mask, pt1)
    s2, d2, m_hh = _pad_edges(two_hop_edge_index, hom_hom_mask, pt2)
    _, _, m_tt = _pad_edges(two_hop_edge_index, het_het_mask, pt2)
    _, _, m_mm = _pad_edges(two_hop_edge_index, mixed_mask, pt2)

    gat1_k = _make_edge_kernel("gat", pt1)
    het1_k = _make_edge_kernel("het", pt1)
    gat2_k = _make_edge_kernel("gat", pt2)
    het2_k = _make_edge_kernel("het", pt2)
    pool_k = _make_pool_kernel()

    w4 = jnp.stack([p["graph_hom"]["Wl"], p["graph_hom"]["Wr"],
                    p["graph_het"]["Wl"], p["graph_het"]["Wr"]])
    h, hl_g, hr_g, hl_h, hr_h = _tc_prep(x, p["ft"]["W"], p["ft"]["b"][None],
                                         w4)

    pm = _perm_matrix()
    gnum1, gden1 = gat1_k(hl_g, hr_g, s1, d1, m_hom,
                          _deinterleave(p["graph_hom"]["a"]))
    (hnum1,) = het1_k(hl_h, hr_h, s1, d1, m_het,
                      _deinterleave(p["graph_het"]["a"]))

    def wstack6(i):
        return jnp.stack([p["hom"][i]["Wl"], p["hom"][i]["Wr"],
                          p["het"][i]["Wl"], p["het"][i]["Wr"],
                          p["mixed"][i]["Wl"], p["mixed"][i]["Wr"]])

    x1, t0, t1, t2, t3, t4, t5 = _tc_combine(
        h, gnum1, gden1, [hnum1], p["graph_hom"]["b"][None],
        [p["graph_het"]["b"][None]], wstack6(0), pm)

    gnum_a, gden_a = gat2_k(t0, t1, s2, d2, m_hh, _deinterleave(p["hom"][0]["a"]))
    (hnum_a,) = het2_k(t2, t3, s2, d2, m_tt, _deinterleave(p["het"][0]["a"]))
    (mnum_a,) = het2_k(t4, t5, s2, d2, m_mm, _deinterleave(p["mixed"][0]["a"]))

    x2, u0, u1, u2, u3, u4, u5 = _tc_combine(
        x1, gnum_a, gden_a, [hnum_a, mnum_a], p["hom"][0]["b"][None],
        [p["het"][0]["b"][None], p["mixed"][0]["b"][None]], wstack6(1), pm)

    gnum_b, gden_b = gat2_k(u0, u1, s2, d2, m_hh, _deinterleave(p["hom"][1]["a"]))
    (hnum_b,) = het2_k(u2, u3, s2, d2, m_tt, _deinterleave(p["het"][1]["a"]))
    (mnum_b,) = het2_k(u4, u5, s2, d2, m_mm, _deinterleave(p["mixed"][1]["a"]))

    (x3,) = _tc_combine(
        x2, gnum_b, gden_b, [hnum_b, mnum_b], p["hom"][1]["b"][None],
        [p["het"][1]["b"][None], p["mixed"][1]["b"][None]], None, pm)

    pools2 = pool_k(x2, batch)
    pools3 = pool_k(x3, batch)

    w3 = p["lin3"]["W"]
    c_out = w3.shape[1]
    w3p = jnp.pad(w3, ((0, 0), (0, H - c_out)))
    b3p = jnp.pad(p["lin3"]["b"], (0, H - c_out),
                  constant_values=-1e30)[None]
    out = _tc_head(pools2, pools3, p["lin1"]["W"], p["lin1"]["b"][None],
                   p["lin2"]["W"], p["lin2"]["b"][None], w3p, b3p)
    return out[:, :c_out]
